# baseline (speedup 1.0000x reference)
import functools
import math

import jax
import jax.numpy as jnp
from jax.experimental import pallas as pl
from jax.experimental.pallas import tpu as pltpu

_R = 128
_W = 512
_GCHUNK = 2048
_INT_MAX = 0x7FFFFFFF
_KC = 16
_NG = 8


def _gather_body(ord_ref, xg_ref, out_ref):
    for r in range(_R):
        idx = ord_ref[r]
        out_ref[r : r + 1, :] = xg_ref[pl.ds(idx, 1), :]


def _kde_body(nch, ni, xs_ref, gr_ref, xts_ref, gc_ref, c0_ref, c1_ref,
              k_ref, out_ref, bits_ref, h_ref):
    b = pl.program_id(0)
    c0 = c0_ref[b]
    c1 = c1_ref[b]
    K = k_ref[0]
    gr = gr_ref[0, :, :]
    xs = xs_ref[0, :, :]

    def fill(j, acc):
        gc = gc_ref[j]
        same = gc == gr
        xt = xts_ref[j]
        d2 = jnp.zeros((_W, _R), jnp.float32)
        for d in range(ni):
            diff = xt[:, d : d + 1] - xs[d : d + 1, :]
            d2 = d2 + diff * diff
        bits = jax.lax.bitcast_convert_type(d2, jnp.int32)
        bits = jnp.where(same, bits, jnp.int32(_INT_MAX))
        bits_ref[j] = bits
        h_ref[j] = (bits >> 16).astype(jnp.int16)
        return acc + same.astype(jnp.int16)

    acc0 = jnp.zeros((_W, _R), jnp.int16)
    acc = jax.lax.fori_loop(c0, c1, fill, acc0)
    cnt = jnp.sum(acc, axis=0, keepdims=True)

    def count16(m16):
        def cs(j, a):
            return a + (h_ref[j] <= m16).astype(jnp.int16)

        a = jax.lax.fori_loop(c0, c1, cs, acc0)
        return jnp.sum(a, axis=0, keepdims=True)

    def step1(_, carry):
        lo, hi = carry
        mid = lo + ((hi - lo) >> 1)
        ge = count16(mid.astype(jnp.int16)) >= _KC
        return jnp.where(ge, lo, mid + 1), jnp.where(ge, mid, hi)

    p, _ = jax.lax.fori_loop(
        0, 15, step1,
        (jnp.zeros((1, _R), jnp.int32), jnp.full((1, _R), 32767, jnp.int32)))

    p16 = p.astype(jnp.int16)

    def comp(j, a):
        bits = bits_ref[j]
        h = h_ref[j]
        lo16 = ((bits & 0xFFFF) - 32768).astype(jnp.int16)
        h_ref[j] = jnp.where(h == p16, lo16, jnp.int16(32767))
        return a + (h < p16).astype(jnp.int16)

    a = jax.lax.fori_loop(c0, c1, comp, acc0)
    k2 = _KC - jnp.sum(a, axis=0, keepdims=True)

    def step2(_, carry):
        lo, hi = carry
        mid = lo + ((hi - lo) >> 1)
        ge = count16(mid.astype(jnp.int16)) >= k2
        return jnp.where(ge, lo, mid + 1), jnp.where(ge, mid, hi)

    l, _ = jax.lax.fori_loop(
        0, 16, step2,
        (jnp.full((1, _R), -32768, jnp.int32),
         jnp.full((1, _R), 32767, jnp.int32)))

    kbits = (p << 16) | (l + 32768)
    kth2 = jax.lax.bitcast_convert_type(kbits, jnp.float32)
    kth = jnp.sqrt(kth2)
    dim = ni - 1
    if dim == 1:
        vol = 2.0 * kth
    elif dim == 2:
        vol = math.pi * kth2
    else:
        vol = (4.0 / 3.0 * math.pi) * (kth2 * kth)
    cf = cnt.astype(jnp.float32)
    kf = K.astype(jnp.float32)
    uniform = 1.0 / jnp.maximum(cf, 1.0)
    vol = jnp.where(cf < kf, 0.0, vol)
    out_ref[0, :, :] = jnp.where(cf < kf, uniform, vol / (kf - 1.0))


def kernel(x, min_t_idx, K):
    N, NI = x.shape
    npin = ((N + _GCHUNK) // _GCHUNK) * _GCHUNK
    np2 = ((N + _NG * (_W - 1) + _GCHUNK - 1) // _GCHUNK) * _GCHUNK
    nch = np2 // _W
    nb2 = np2 // _R

    g = min_t_idx.astype(jnp.int32)
    gids = jnp.arange(_NG, dtype=jnp.int32)
    oh = g[None, :] == gids[:, None]
    counts = jnp.sum(oh.astype(jnp.int32), axis=1)
    padded = ((counts + _W - 1) // _W) * _W
    seg_end = jnp.cumsum(padded)
    seg_start = seg_end - padded
    csum = jnp.cumsum(oh.astype(jnp.int32), axis=1)
    rank = jnp.sum(jnp.where(oh, csum - 1, 0), axis=0)
    pos = seg_start[g] + rank
    order = jnp.full((np2,), N, jnp.int32).at[pos].set(
        jnp.arange(N, dtype=jnp.int32))
    inv = jnp.concatenate(
        [pos, jnp.full((npin - N,), np2 - 1, jnp.int32)]).astype(jnp.int32)

    blk = jnp.arange(nb2, dtype=jnp.int32) * _R
    gb = jnp.searchsorted(seg_end, blk, side="right")
    gbc = jnp.minimum(gb, _NG - 1)
    c0 = jnp.where(gb < _NG, seg_start[gbc] // _W, 0).astype(jnp.int32)
    c1 = jnp.where(gb < _NG, seg_end[gbc] // _W, 0).astype(jnp.int32)

    xpad = jnp.zeros((npin, NI), jnp.float32).at[:N].set(x.astype(jnp.float32))
    gpad = jnp.full((npin,), -1, jnp.int32).at[:N].set(g)
    xg = jnp.concatenate([xpad, gpad[:, None].astype(jnp.float32)], axis=1)

    xsg = pl.pallas_call(
        _gather_body,
        grid=(nb2,),
        in_specs=[
            pl.BlockSpec((_R,), lambda i: (i,), memory_space=pltpu.SMEM),
            pl.BlockSpec((npin, NI + 1), lambda i: (0, 0)),
        ],
        out_specs=pl.BlockSpec((_R, NI + 1), lambda i: (i, 0)),
        out_shape=jax.ShapeDtypeStruct((np2, NI + 1), jnp.float32),
    )(order, xg)

    xs = xsg[:, :NI]
    gs = xsg[:, NI].astype(jnp.int32)
    xs_t = jnp.transpose(xs.reshape(nb2, _R, NI), (0, 2, 1))
    gr_t = gs.reshape(nb2, 1, _R)
    xts_c = xs.reshape(nch, _W, NI)
    gc_c = gs.reshape(nch, _W, 1)

    ps = pl.pallas_call(
        functools.partial(_kde_body, nch, NI),
        grid=(nb2,),
        in_specs=[
            pl.BlockSpec((1, NI, _R), lambda i: (i, 0, 0)),
            pl.BlockSpec((1, 1, _R), lambda i: (i, 0, 0)),
            pl.BlockSpec((nch, _W, NI), lambda i: (0, 0, 0)),
            pl.BlockSpec((nch, _W, 1), lambda i: (0, 0, 0)),
            pl.BlockSpec(memory_space=pltpu.SMEM),
            pl.BlockSpec(memory_space=pltpu.SMEM),
            pl.BlockSpec(memory_space=pltpu.SMEM),
        ],
        out_specs=pl.BlockSpec((1, 1, _R), lambda i: (i, 0, 0)),
        out_shape=jax.ShapeDtypeStruct((nb2, 1, _R), jnp.float32),
        scratch_shapes=[pltpu.VMEM((nch, _W, _R), jnp.int32),
                        pltpu.VMEM((nch, _W, _R), jnp.int16)],
    )(xs_t, gr_t, xts_c, gc_c, c0, c1, jnp.full((1,), K, jnp.int32))
    ps = ps.reshape(np2, 1)

    pout = pl.pallas_call(
        _gather_body,
        grid=(npin // _R,),
        in_specs=[
            pl.BlockSpec((_R,), lambda i: (i,), memory_space=pltpu.SMEM),
            pl.BlockSpec((np2, 1), lambda i: (0, 0)),
        ],
        out_specs=pl.BlockSpec((_R, 1), lambda i: (i, 0)),
        out_shape=jax.ShapeDtypeStruct((npin, 1), jnp.float32),
    )(inv, ps)

    return jax.lax.stop_gradient(pout[:N, 0])


# speedup vs baseline: 1.2465x; 1.2465x over previous
import functools
import math

import jax
import jax.numpy as jnp
from jax.experimental import pallas as pl
from jax.experimental.pallas import tpu as pltpu

_R = 128
_W = 512
_GCHUNK = 2048
_INT_MAX = 0x7FFFFFFF
_KC = 16
_NG = 8


def _gather_body(ord_ref, xg_ref, out_ref):
    for r in range(_R):
        idx = ord_ref[r]
        out_ref[r : r + 1, :] = xg_ref[pl.ds(idx, 1), :]


def _kde_body(nch, ni, xs_ref, gr_ref, xts_ref, gc_ref, c0_ref, c1_ref,
              k_ref, out_ref, bits_ref, h_ref):
    b = pl.program_id(0)
    c0 = c0_ref[b]
    c1 = c1_ref[b]
    K = k_ref[0]
    gr = gr_ref[:, :]
    xs = xs_ref[:, :]

    def fill(j, acc):
        gc = gc_ref[j]
        same = gr == gc
        xt = xts_ref[j]
        d2 = jnp.zeros((_R, _W), jnp.float32)
        for d in range(ni):
            diff = xs[:, d : d + 1] - xt[d : d + 1, :]
            d2 = d2 + diff * diff
        bits = jax.lax.bitcast_convert_type(d2, jnp.int32)
        bits = jnp.where(same, bits, jnp.int32(_INT_MAX))
        bits_ref[j] = bits
        h_ref[j] = (bits >> 16).astype(jnp.int16)
        return acc + same.astype(jnp.int16)

    acc0 = jnp.zeros((_R, _W), jnp.int16)
    acc = jax.lax.fori_loop(c0, c1, fill, acc0)
    cnt = jnp.sum(acc, axis=1, keepdims=True)

    def count16(m16):
        def cs(j, a):
            return a + (h_ref[j] <= m16).astype(jnp.int16)

        a = jax.lax.fori_loop(c0, c1, cs, acc0)
        return jnp.sum(a, axis=1, keepdims=True)

    def step1(_, carry):
        lo, hi = carry
        mid = lo + ((hi - lo) >> 1)
        ge = count16(mid.astype(jnp.int16)) >= _KC
        return jnp.where(ge, lo, mid + 1), jnp.where(ge, mid, hi)

    p, _ = jax.lax.fori_loop(
        0, 15, step1,
        (jnp.zeros((_R, 1), jnp.int32), jnp.full((_R, 1), 32767, jnp.int32)))

    p16 = p.astype(jnp.int16)

    def comp(j, a):
        bits = bits_ref[j]
        h = h_ref[j]
        lo16 = ((bits & 0xFFFF) - 32768).astype(jnp.int16)
        h_ref[j] = jnp.where(h == p16, lo16, jnp.int16(32767))
        return a + (h < p16).astype(jnp.int16)

    a = jax.lax.fori_loop(c0, c1, comp, acc0)
    k2 = _KC - jnp.sum(a, axis=1, keepdims=True)

    def step2(_, carry):
        lo, hi = carry
        mid = lo + ((hi - lo) >> 1)
        ge = count16(mid.astype(jnp.int16)) >= k2
        return jnp.where(ge, lo, mid + 1), jnp.where(ge, mid, hi)

    l, _ = jax.lax.fori_loop(
        0, 16, step2,
        (jnp.full((_R, 1), -32768, jnp.int32),
         jnp.full((_R, 1), 32767, jnp.int32)))

    kbits = (p << 16) | (l + 32768)
    kth2 = jax.lax.bitcast_convert_type(kbits, jnp.float32)
    kth = jnp.sqrt(kth2)
    dim = ni - 1
    if dim == 1:
        vol = 2.0 * kth
    elif dim == 2:
        vol = math.pi * kth2
    else:
        vol = (4.0 / 3.0 * math.pi) * (kth2 * kth)
    cf = cnt.astype(jnp.float32)
    kf = K.astype(jnp.float32)
    uniform = 1.0 / jnp.maximum(cf, 1.0)
    vol = jnp.where(cf < kf, 0.0, vol)
    out_ref[:, :] = jnp.where(cf < kf, uniform, vol / (kf - 1.0))


def kernel(x, min_t_idx, K):
    N, NI = x.shape
    npin = ((N + _GCHUNK) // _GCHUNK) * _GCHUNK
    np2 = ((N + _NG * (_W - 1) + _GCHUNK - 1) // _GCHUNK) * _GCHUNK
    nch = np2 // _W
    nb2 = np2 // _R

    g = min_t_idx.astype(jnp.int32)
    gids = jnp.arange(_NG, dtype=jnp.int32)
    oh = g[None, :] == gids[:, None]
    counts = jnp.sum(oh.astype(jnp.int32), axis=1)
    padded = ((counts + _W - 1) // _W) * _W
    seg_end = jnp.cumsum(padded)
    seg_start = seg_end - padded
    csum = jnp.cumsum(oh.astype(jnp.int32), axis=1)
    rank = jnp.sum(jnp.where(oh, csum - 1, 0), axis=0)
    pos = seg_start[g] + rank
    order = jnp.full((np2,), N, jnp.int32).at[pos].set(
        jnp.arange(N, dtype=jnp.int32))
    inv = jnp.concatenate(
        [pos, jnp.full((npin - N,), np2 - 1, jnp.int32)]).astype(jnp.int32)

    blk = jnp.arange(nb2, dtype=jnp.int32) * _R
    gb = jnp.searchsorted(seg_end, blk, side="right")
    gbc = jnp.minimum(gb, _NG - 1)
    c0 = jnp.where(gb < _NG, seg_start[gbc] // _W, 0).astype(jnp.int32)
    c1 = jnp.where(gb < _NG, seg_end[gbc] // _W, 0).astype(jnp.int32)

    xpad = jnp.zeros((npin, NI), jnp.float32).at[:N].set(x.astype(jnp.float32))
    gpad = jnp.full((npin,), -1, jnp.int32).at[:N].set(g)
    xg = jnp.concatenate([xpad, gpad[:, None].astype(jnp.float32)], axis=1)

    xsg = pl.pallas_call(
        _gather_body,
        grid=(nb2,),
        in_specs=[
            pl.BlockSpec((_R,), lambda i: (i,), memory_space=pltpu.SMEM),
            pl.BlockSpec((npin, NI + 1), lambda i: (0, 0)),
        ],
        out_specs=pl.BlockSpec((_R, NI + 1), lambda i: (i, 0)),
        out_shape=jax.ShapeDtypeStruct((np2, NI + 1), jnp.float32),
    )(order, xg)

    xs = xsg[:, :NI]
    gs = xsg[:, NI].astype(jnp.int32)
    gr_s = gs[:, None]
    gc_s = gs.reshape(nch, 1, _W)
    xts = jnp.transpose(xs.reshape(nch, _W, NI), (0, 2, 1))

    ps = pl.pallas_call(
        functools.partial(_kde_body, nch, NI),
        grid=(nb2,),
        in_specs=[
            pl.BlockSpec((_R, NI), lambda i: (i, 0)),
            pl.BlockSpec((_R, 1), lambda i: (i, 0)),
            pl.BlockSpec((nch, NI, _W), lambda i: (0, 0, 0)),
            pl.BlockSpec((nch, 1, _W), lambda i: (0, 0, 0)),
            pl.BlockSpec(memory_space=pltpu.SMEM),
            pl.BlockSpec(memory_space=pltpu.SMEM),
            pl.BlockSpec(memory_space=pltpu.SMEM),
        ],
        out_specs=pl.BlockSpec((_R, 1), lambda i: (i, 0)),
        out_shape=jax.ShapeDtypeStruct((np2, 1), jnp.float32),
        scratch_shapes=[pltpu.VMEM((nch, _R, _W), jnp.int32),
                        pltpu.VMEM((nch, _R, _W), jnp.int16)],
    )(xs, gr_s, xts, gc_s, c0, c1, jnp.full((1,), K, jnp.int32))

    pout = pl.pallas_call(
        _gather_body,
        grid=(npin // _R,),
        in_specs=[
            pl.BlockSpec((_R,), lambda i: (i,), memory_space=pltpu.SMEM),
            pl.BlockSpec((np2, 1), lambda i: (0, 0)),
        ],
        out_specs=pl.BlockSpec((_R, 1), lambda i: (i, 0)),
        out_shape=jax.ShapeDtypeStruct((npin, 1), jnp.float32),
    )(inv, ps)

    return jax.lax.stop_gradient(pout[:N, 0])


# speedup vs baseline: 1.3132x; 1.0535x over previous
import functools
import math

import jax
import jax.numpy as jnp
from jax.experimental import pallas as pl
from jax.experimental.pallas import tpu as pltpu

_R = 256
_W = 512
_GCHUNK = 2048
_INT_MAX = 0x7FFFFFFF
_KC = 16
_NG = 8


def _gather_body(ord_ref, xg_ref, out_ref):
    for r in range(_R):
        idx = ord_ref[r]
        out_ref[r : r + 1, :] = xg_ref[pl.ds(idx, 1), :]


def _kde_body(nch, ni, xs_ref, gr_ref, xts_ref, gc_ref, c0_ref, c1_ref,
              k_ref, out_ref, bits_ref, h_ref):
    b = pl.program_id(0)
    c0 = c0_ref[b]
    c1 = c1_ref[b]
    K = k_ref[0]
    gr = gr_ref[:, :]
    xs = xs_ref[:, :]

    def fill(j, acc):
        gc = gc_ref[j]
        same = gr == gc
        xt = xts_ref[j]
        d2 = jnp.zeros((_R, _W), jnp.float32)
        for d in range(ni):
            diff = xs[:, d : d + 1] - xt[d : d + 1, :]
            d2 = d2 + diff * diff
        bits = jax.lax.bitcast_convert_type(d2, jnp.int32)
        bits = jnp.where(same, bits, jnp.int32(_INT_MAX))
        bits_ref[j] = bits
        h_ref[j] = (bits >> 16).astype(jnp.int16)
        return acc + same.astype(jnp.int16)

    acc0 = jnp.zeros((_R, _W), jnp.int16)
    acc = jax.lax.fori_loop(c0, c1, fill, acc0)
    cnt = jnp.sum(acc, axis=1, keepdims=True)

    def count16(m16):
        def cs(j, a):
            return a + (h_ref[j] <= m16).astype(jnp.int16)

        a = jax.lax.fori_loop(c0, c1, cs, acc0)
        return jnp.sum(a, axis=1, keepdims=True)

    def step1(_, carry):
        lo, hi = carry
        mid = lo + ((hi - lo) >> 1)
        ge = count16(mid.astype(jnp.int16)) >= _KC
        return jnp.where(ge, lo, mid + 1), jnp.where(ge, mid, hi)

    p, _ = jax.lax.fori_loop(
        0, 15, step1,
        (jnp.zeros((_R, 1), jnp.int32), jnp.full((_R, 1), 32767, jnp.int32)))

    p16 = p.astype(jnp.int16)

    def comp(j, a):
        bits = bits_ref[j]
        h = h_ref[j]
        lo16 = ((bits & 0xFFFF) - 32768).astype(jnp.int16)
        h_ref[j] = jnp.where(h == p16, lo16, jnp.int16(32767))
        return a + (h < p16).astype(jnp.int16)

    a = jax.lax.fori_loop(c0, c1, comp, acc0)
    k2 = _KC - jnp.sum(a, axis=1, keepdims=True)

    def step2(_, carry):
        lo, hi = carry
        mid = lo + ((hi - lo) >> 1)
        ge = count16(mid.astype(jnp.int16)) >= k2
        return jnp.where(ge, lo, mid + 1), jnp.where(ge, mid, hi)

    l, _ = jax.lax.fori_loop(
        0, 16, step2,
        (jnp.full((_R, 1), -32768, jnp.int32),
         jnp.full((_R, 1), 32767, jnp.int32)))

    kbits = (p << 16) | (l + 32768)
    kth2 = jax.lax.bitcast_convert_type(kbits, jnp.float32)
    kth = jnp.sqrt(kth2)
    dim = ni - 1
    if dim == 1:
        vol = 2.0 * kth
    elif dim == 2:
        vol = math.pi * kth2
    else:
        vol = (4.0 / 3.0 * math.pi) * (kth2 * kth)
    cf = cnt.astype(jnp.float32)
    kf = K.astype(jnp.float32)
    uniform = 1.0 / jnp.maximum(cf, 1.0)
    vol = jnp.where(cf < kf, 0.0, vol)
    out_ref[:, :] = jnp.where(cf < kf, uniform, vol / (kf - 1.0))


def kernel(x, min_t_idx, K):
    N, NI = x.shape
    npin = ((N + _GCHUNK) // _GCHUNK) * _GCHUNK
    np2 = ((N + _NG * (_W - 1) + _GCHUNK - 1) // _GCHUNK) * _GCHUNK
    nch = np2 // _W
    nb2 = np2 // _R

    g = min_t_idx.astype(jnp.int32)
    gids = jnp.arange(_NG, dtype=jnp.int32)
    oh = g[None, :] == gids[:, None]
    counts = jnp.sum(oh.astype(jnp.int32), axis=1)
    padded = ((counts + _W - 1) // _W) * _W
    seg_end = jnp.cumsum(padded)
    seg_start = seg_end - padded
    csum = jnp.cumsum(oh.astype(jnp.int32), axis=1)
    rank = jnp.sum(jnp.where(oh, csum - 1, 0), axis=0)
    pos = seg_start[g] + rank
    order = jnp.full((np2,), N, jnp.int32).at[pos].set(
        jnp.arange(N, dtype=jnp.int32))
    inv = jnp.concatenate(
        [pos, jnp.full((npin - N,), np2 - 1, jnp.int32)]).astype(jnp.int32)

    blk = jnp.arange(nb2, dtype=jnp.int32) * _R
    gb = jnp.searchsorted(seg_end, blk, side="right")
    gbc = jnp.minimum(gb, _NG - 1)
    c0 = jnp.where(gb < _NG, seg_start[gbc] // _W, 0).astype(jnp.int32)
    c1 = jnp.where(gb < _NG, seg_end[gbc] // _W, 0).astype(jnp.int32)

    xpad = jnp.zeros((npin, NI), jnp.float32).at[:N].set(x.astype(jnp.float32))
    gpad = jnp.full((npin,), -1, jnp.int32).at[:N].set(g)
    xg = jnp.concatenate([xpad, gpad[:, None].astype(jnp.float32)], axis=1)

    xsg = pl.pallas_call(
        _gather_body,
        grid=(nb2,),
        in_specs=[
            pl.BlockSpec((_R,), lambda i: (i,), memory_space=pltpu.SMEM),
            pl.BlockSpec((npin, NI + 1), lambda i: (0, 0)),
        ],
        out_specs=pl.BlockSpec((_R, NI + 1), lambda i: (i, 0)),
        out_shape=jax.ShapeDtypeStruct((np2, NI + 1), jnp.float32),
    )(order, xg)

    xs = xsg[:, :NI]
    gs = xsg[:, NI].astype(jnp.int32)
    gr_s = gs[:, None]
    gc_s = gs.reshape(nch, 1, _W)
    xts = jnp.transpose(xs.reshape(nch, _W, NI), (0, 2, 1))

    ps = pl.pallas_call(
        functools.partial(_kde_body, nch, NI),
        grid=(nb2,),
        in_specs=[
            pl.BlockSpec((_R, NI), lambda i: (i, 0)),
            pl.BlockSpec((_R, 1), lambda i: (i, 0)),
            pl.BlockSpec((nch, NI, _W), lambda i: (0, 0, 0)),
            pl.BlockSpec((nch, 1, _W), lambda i: (0, 0, 0)),
            pl.BlockSpec(memory_space=pltpu.SMEM),
            pl.BlockSpec(memory_space=pltpu.SMEM),
            pl.BlockSpec(memory_space=pltpu.SMEM),
        ],
        out_specs=pl.BlockSpec((_R, 1), lambda i: (i, 0)),
        out_shape=jax.ShapeDtypeStruct((np2, 1), jnp.float32),
        scratch_shapes=[pltpu.VMEM((nch, _R, _W), jnp.int32),
                        pltpu.VMEM((nch, _R, _W), jnp.int16)],
    )(xs, gr_s, xts, gc_s, c0, c1, jnp.full((1,), K, jnp.int32))

    pout = pl.pallas_call(
        _gather_body,
        grid=(npin // _R,),
        in_specs=[
            pl.BlockSpec((_R,), lambda i: (i,), memory_space=pltpu.SMEM),
            pl.BlockSpec((np2, 1), lambda i: (0, 0)),
        ],
        out_specs=pl.BlockSpec((_R, 1), lambda i: (i, 0)),
        out_shape=jax.ShapeDtypeStruct((npin, 1), jnp.float32),
    )(inv, ps)

    return jax.lax.stop_gradient(pout[:N, 0])


# speedup vs baseline: 1.5886x; 1.2097x over previous
import functools
import math

import jax
import jax.numpy as jnp
from jax.experimental import pallas as pl
from jax.experimental.pallas import tpu as pltpu

_R = 256
_W = 512
_GCHUNK = 2048
_INT_MAX = 0x7FFFFFFF
_KC = 16
_NG = 8


def _gather_body(ord_ref, xg_ref, out_ref):
    for r in range(_R):
        idx = ord_ref[r]
        out_ref[r : r + 1, :] = xg_ref[pl.ds(idx, 1), :]


def _kde_body(nch, ni, xs_ref, gr_ref, xts_ref, gc_ref, c0_ref, c1_ref,
              k_ref, out_ref, bits_ref, h_ref):
    b = pl.program_id(0)
    c0 = c0_ref[b]
    c1 = c1_ref[b]
    K = k_ref[0]
    gr = gr_ref[:, :]
    xs = xs_ref[:, :]

    def fill(j, acc):
        gc = gc_ref[j]
        same = gr == gc
        xt = xts_ref[j]
        d2 = jnp.zeros((_R, _W), jnp.float32)
        for d in range(ni):
            diff = xs[:, d : d + 1] - xt[d : d + 1, :]
            d2 = d2 + diff * diff
        bits = jax.lax.bitcast_convert_type(d2, jnp.int32)
        bits = jnp.where(same, bits, jnp.int32(_INT_MAX))
        bits_ref[j] = bits
        h_ref[j] = (bits >> 16).astype(jnp.int16)
        return acc + same.astype(jnp.int16)

    acc0 = jnp.zeros((_R, _W), jnp.int16)
    acc = jax.lax.fori_loop(c0, c1, fill, acc0)
    cnt = jnp.sum(acc, axis=1, keepdims=True)

    def count16(m16):
        def cs(j, a):
            return a + (h_ref[j] <= m16).astype(jnp.int16)

        a = jax.lax.fori_loop(c0, c1, cs, acc0)
        return jnp.sum(a, axis=1, keepdims=True)

    def step1(_, carry):
        lo, hi = carry
        mid = lo + ((hi - lo) >> 1)
        ge = count16(mid.astype(jnp.int16)) >= _KC
        return jnp.where(ge, lo, mid + 1), jnp.where(ge, mid, hi)

    p, _ = jax.lax.fori_loop(
        0, 15, step1,
        (jnp.zeros((_R, 1), jnp.int32), jnp.full((_R, 1), 32767, jnp.int32)))

    p16 = p.astype(jnp.int16)
    maxw = jnp.full((_R, _W), _INT_MAX, jnp.int32)

    def comp2(j, carry):
        a, m = carry
        a = a + (h_ref[j] < p16).astype(jnp.int16)
        m = jnp.minimum(
            m, jnp.where(h_ref[j] == p16, bits_ref[j], jnp.int32(_INT_MAX)))
        return a, m

    a, m = jax.lax.fori_loop(c0, c1, comp2, (acc0, maxw))
    c_lt = jnp.sum(a, axis=1, keepdims=True)
    cur = jnp.min(m, axis=1, keepdims=True)

    def ceq(j, acc):
        return acc + (bits_ref[j] == cur).astype(jnp.int16)

    r = c_lt + jnp.sum(jax.lax.fori_loop(c0, c1, ceq, acc0),
                       axis=1, keepdims=True)
    nd0 = jnp.sum((r < _KC).astype(jnp.int32)) * (c1 - c0)

    def w_cond(carry):
        return carry[2] > 0

    def w_body(carry):
        cur, r, _ = carry

        def nxt_loop(j, mm):
            b = bits_ref[j]
            ok = (h_ref[j] == p16) & (b > cur)
            return jnp.minimum(mm, jnp.where(ok, b, jnp.int32(_INT_MAX)))

        nxt = jnp.min(jax.lax.fori_loop(c0, c1, nxt_loop, maxw),
                      axis=1, keepdims=True)

        def ceq2(j, acc):
            return acc + (bits_ref[j] == nxt).astype(jnp.int16)

        c = jnp.sum(jax.lax.fori_loop(c0, c1, ceq2, acc0),
                    axis=1, keepdims=True)
        done = r >= _KC
        cur2 = jnp.where(done, cur, nxt)
        r2 = jnp.where(done, r, r + c)
        return cur2, r2, jnp.sum((r2 < _KC).astype(jnp.int32))

    cur, _, _ = jax.lax.while_loop(w_cond, w_body, (cur, r, nd0))
    kbits = cur
    kth2 = jax.lax.bitcast_convert_type(kbits, jnp.float32)
    kth = jnp.sqrt(kth2)
    dim = ni - 1
    if dim == 1:
        vol = 2.0 * kth
    elif dim == 2:
        vol = math.pi * kth2
    else:
        vol = (4.0 / 3.0 * math.pi) * (kth2 * kth)
    cf = cnt.astype(jnp.float32)
    kf = K.astype(jnp.float32)
    uniform = 1.0 / jnp.maximum(cf, 1.0)
    vol = jnp.where(cf < kf, 0.0, vol)
    out_ref[:, :] = jnp.where(cf < kf, uniform, vol / (kf - 1.0))


def kernel(x, min_t_idx, K):
    N, NI = x.shape
    npin = ((N + _GCHUNK) // _GCHUNK) * _GCHUNK
    np2 = ((N + _NG * (_W - 1) + _GCHUNK - 1) // _GCHUNK) * _GCHUNK
    nch = np2 // _W
    nb2 = np2 // _R

    g = min_t_idx.astype(jnp.int32)
    gids = jnp.arange(_NG, dtype=jnp.int32)
    oh = g[None, :] == gids[:, None]
    counts = jnp.sum(oh.astype(jnp.int32), axis=1)
    padded = ((counts + _W - 1) // _W) * _W
    seg_end = jnp.cumsum(padded)
    seg_start = seg_end - padded
    csum = jnp.cumsum(oh.astype(jnp.int32), axis=1)
    rank = jnp.sum(jnp.where(oh, csum - 1, 0), axis=0)
    pos = seg_start[g] + rank
    order = jnp.full((np2,), N, jnp.int32).at[pos].set(
        jnp.arange(N, dtype=jnp.int32))
    inv = jnp.concatenate(
        [pos, jnp.full((npin - N,), np2 - 1, jnp.int32)]).astype(jnp.int32)

    blk = jnp.arange(nb2, dtype=jnp.int32) * _R
    gb = jnp.searchsorted(seg_end, blk, side="right")
    gbc = jnp.minimum(gb, _NG - 1)
    c0 = jnp.where(gb < _NG, seg_start[gbc] // _W, 0).astype(jnp.int32)
    c1 = jnp.where(gb < _NG, seg_end[gbc] // _W, 0).astype(jnp.int32)

    xpad = jnp.zeros((npin, NI), jnp.float32).at[:N].set(x.astype(jnp.float32))
    gpad = jnp.full((npin,), -1, jnp.int32).at[:N].set(g)
    xg = jnp.concatenate([xpad, gpad[:, None].astype(jnp.float32)], axis=1)

    xsg = pl.pallas_call(
        _gather_body,
        grid=(nb2,),
        in_specs=[
            pl.BlockSpec((_R,), lambda i: (i,), memory_space=pltpu.SMEM),
            pl.BlockSpec((npin, NI + 1), lambda i: (0, 0)),
        ],
        out_specs=pl.BlockSpec((_R, NI + 1), lambda i: (i, 0)),
        out_shape=jax.ShapeDtypeStruct((np2, NI + 1), jnp.float32),
    )(order, xg)

    xs = xsg[:, :NI]
    gs = xsg[:, NI].astype(jnp.int32)
    gr_s = gs[:, None]
    gc_s = gs.reshape(nch, 1, _W)
    xts = jnp.transpose(xs.reshape(nch, _W, NI), (0, 2, 1))

    ps = pl.pallas_call(
        functools.partial(_kde_body, nch, NI),
        grid=(nb2,),
        in_specs=[
            pl.BlockSpec((_R, NI), lambda i: (i, 0)),
            pl.BlockSpec((_R, 1), lambda i: (i, 0)),
            pl.BlockSpec((nch, NI, _W), lambda i: (0, 0, 0)),
            pl.BlockSpec((nch, 1, _W), lambda i: (0, 0, 0)),
            pl.BlockSpec(memory_space=pltpu.SMEM),
            pl.BlockSpec(memory_space=pltpu.SMEM),
            pl.BlockSpec(memory_space=pltpu.SMEM),
        ],
        out_specs=pl.BlockSpec((_R, 1), lambda i: (i, 0)),
        out_shape=jax.ShapeDtypeStruct((np2, 1), jnp.float32),
        scratch_shapes=[pltpu.VMEM((nch, _R, _W), jnp.int32),
                        pltpu.VMEM((nch, _R, _W), jnp.int16)],
    )(xs, gr_s, xts, gc_s, c0, c1, jnp.full((1,), K, jnp.int32))

    pout = pl.pallas_call(
        _gather_body,
        grid=(npin // _R,),
        in_specs=[
            pl.BlockSpec((_R,), lambda i: (i,), memory_space=pltpu.SMEM),
            pl.BlockSpec((np2, 1), lambda i: (0, 0)),
        ],
        out_specs=pl.BlockSpec((_R, 1), lambda i: (i, 0)),
        out_shape=jax.ShapeDtypeStruct((npin, 1), jnp.float32),
    )(inv, ps)

    return jax.lax.stop_gradient(pout[:N, 0])
